# baseline (speedup 1.0000x reference)
import functools

import jax
import jax.numpy as jnp
from jax import lax
from jax.experimental import pallas as pl
from jax.experimental.pallas import tpu as pltpu
from jax.experimental.pallas import tpu_sc as plsc


def _make_sc_kernel(q, k, d):
    info = plsc.get_sparse_core_info()
    nw = info.num_cores * info.num_subcores
    rows_per_w = q // nw
    kd = k * d
    mesh = plsc.VectorSubcoreMesh(core_axis_name="c", subcore_axis_name="s")

    @functools.partial(
        pl.kernel,
        mesh=mesh,
        out_type=jax.ShapeDtypeStruct((q, kd), jnp.float32),
        scratch_types=[
            pltpu.VMEM((rows_per_w * d,), jnp.float32),
            pltpu.VMEM((kd,), jnp.float32),
            pltpu.VMEM((kd,), jnp.float32),
        ],
    )
    def sck(x_hbm, y_hbm, out_hbm, x_v, y_v, row_v):
        wid = lax.axis_index("s") * info.num_cores + lax.axis_index("c")
        base = wid * rows_per_w
        pltpu.sync_copy(x_hbm.at[pl.ds(base * d, rows_per_w * d)], x_v)
        pltpu.sync_copy(y_hbm, y_v)

        def do_row(r, _):
            xi = x_v[pl.ds(r * d, d)]

            def do_col(j, _):
                row_v[pl.ds(j * d, d)] = xi + y_v[pl.ds(j * d, d)]
                return 0

            lax.fori_loop(0, k, do_col, 0)
            pltpu.sync_copy(row_v, out_hbm.at[base + r])
            return 0

        lax.fori_loop(0, rows_per_w, do_row, 0)

    return sck


def kernel(query_size, key_size, x_emb, y_emb):
    q, d = x_emb.shape
    k, _ = y_emb.shape
    x_eff = jnp.take(x_emb, jnp.arange(q) + (query_size - q), axis=0)
    y_eff = jnp.take(y_emb, jnp.arange(k) + (key_size - k), axis=0)
    out2 = _make_sc_kernel(q, k, d)(x_eff.reshape(q * d), y_eff.reshape(k * d))
    return out2.reshape(q, k, d)


# speedup vs baseline: 8.4145x; 8.4145x over previous
import jax
import jax.numpy as jnp
from jax.experimental import pallas as pl


def _outer_sum_kernel(x_ref, yt_ref, o_ref):
    o_ref[...] = x_ref[...][:, :, None] + yt_ref[...][None, :, :]


def kernel(query_size, key_size, x_emb, y_emb):
    q, d = x_emb.shape
    k, _ = y_emb.shape
    x_eff = jnp.take(x_emb, jnp.arange(q) + (query_size - q), axis=0)
    y_eff = jnp.take(y_emb, jnp.arange(k) + (key_size - k), axis=0)

    yt = y_eff.T
    bq = 64
    out3 = pl.pallas_call(
        _outer_sum_kernel,
        grid=(q // bq,),
        in_specs=[
            pl.BlockSpec((bq, d), lambda i: (i, 0)),
            pl.BlockSpec((d, k), lambda i: (0, 0)),
        ],
        out_specs=pl.BlockSpec((bq, d, k), lambda i: (i, 0, 0)),
        out_shape=jax.ShapeDtypeStruct((q, d, k), x_emb.dtype),
    )(x_eff, yt)
    return jnp.transpose(out3, (0, 2, 1))


# speedup vs baseline: 8.5734x; 1.0189x over previous
import jax
import jax.numpy as jnp
from jax.experimental import pallas as pl


def _outer_sum_kernel(x_ref, yt_ref, o_ref):
    o_ref[...] = x_ref[...][:, :, None] + yt_ref[...][None, :, :]


def kernel(query_size, key_size, x_emb, y_emb):
    q, d = x_emb.shape
    k, _ = y_emb.shape
    x_eff = jnp.take(x_emb, jnp.arange(q) + (query_size - q), axis=0)
    y_eff = jnp.take(y_emb, jnp.arange(k) + (key_size - k), axis=0)

    yt = y_eff.T
    bq = 128
    out3 = pl.pallas_call(
        _outer_sum_kernel,
        grid=(q // bq,),
        in_specs=[
            pl.BlockSpec((bq, d), lambda i: (i, 0)),
            pl.BlockSpec((d, k), lambda i: (0, 0)),
        ],
        out_specs=pl.BlockSpec((bq, d, k), lambda i: (i, 0, 0)),
        out_shape=jax.ShapeDtypeStruct((q, d, k), x_emb.dtype),
    )(x_eff, yt)
    return jnp.transpose(out3, (0, 2, 1))


# speedup vs baseline: 8.7055x; 1.0154x over previous
import jax
import jax.numpy as jnp
from jax.experimental import pallas as pl


def _outer_sum_kernel(x_ref, yt_ref, o_ref):
    o_ref[...] = x_ref[...][:, :, None] + yt_ref[...][None, :, :]


def kernel(query_size, key_size, x_emb, y_emb):
    q, d = x_emb.shape
    k, _ = y_emb.shape
    x_eff = jnp.take(x_emb, jnp.arange(q) + (query_size - q), axis=0)
    y_eff = jnp.take(y_emb, jnp.arange(k) + (key_size - k), axis=0)

    yt = y_eff.T
    bq = 32
    out3 = pl.pallas_call(
        _outer_sum_kernel,
        grid=(q // bq,),
        in_specs=[
            pl.BlockSpec((bq, d), lambda i: (i, 0)),
            pl.BlockSpec((d, k), lambda i: (0, 0)),
        ],
        out_specs=pl.BlockSpec((bq, d, k), lambda i: (i, 0, 0)),
        out_shape=jax.ShapeDtypeStruct((q, d, k), x_emb.dtype),
    )(x_eff, yt)
    return jnp.transpose(out3, (0, 2, 1))
